# baseline (speedup 1.0000x reference)
import functools

import jax
import jax.numpy as jnp
from jax import lax
from jax.experimental import pallas as pl
from jax.experimental.pallas import tpu as pltpu
from jax.experimental.pallas import tpu_sc as plsc

N_NODES = 1000000
EMBED = 64
B = 16384


def _sc_gather(table, idx):
    info = plsc.get_sparse_core_info()
    nw = info.num_cores * info.num_subcores
    b_per_w = B // nw
    mesh = plsc.VectorSubcoreMesh(core_axis_name="c", subcore_axis_name="s")

    @functools.partial(
        pl.kernel,
        mesh=mesh,
        compiler_params=pltpu.CompilerParams(use_tc_tiling_on_sc=False),
        out_type=jax.ShapeDtypeStruct((B, EMBED), jnp.float32),
        scratch_types=[
            pltpu.VMEM((b_per_w,), jnp.int32),
            pltpu.VMEM((b_per_w, EMBED), jnp.float32),
            pltpu.SemaphoreType.DMA,
        ],
    )
    def k(table_hbm, idx_hbm, out_hbm, idx_v, rows_v, sem):
        wid = lax.axis_index("s") * info.num_cores + lax.axis_index("c")
        base = wid * b_per_w
        pltpu.sync_copy(idx_hbm.at[pl.ds(base, b_per_w)], idx_v)

        @pl.loop(0, b_per_w, step=16)
        def _issue(i0):
            vec = idx_v[pl.ds(i0, 16)]
            for j in range(16):
                pltpu.async_copy(
                    table_hbm.at[pl.ds(vec[j], 1), :],
                    rows_v.at[pl.ds(i0 + j, 1), :],
                    sem,
                )

        pltpu.make_async_copy(
            table_hbm.at[pl.ds(0, b_per_w), :], rows_v, sem
        ).wait()
        pltpu.sync_copy(rows_v, out_hbm.at[pl.ds(base, b_per_w)])

    return k(table, idx)


def _tc_dense(emb, sig_t, fc_w, fc_b, interpret=False):
    blk = 2048

    def body(emb_ref, sig_ref, w_ref, b_ref, out_ref):
        w = w_ref[...]
        acc = jnp.sum(emb_ref[...] * w[0, :EMBED][None, :], axis=1)
        st = sig_ref[...]
        for k in range(3):
            wk = w[0, (k + 1) * EMBED:(k + 2) * EMBED]
            acc = acc + jnp.sum(st[k] * wk[:, None], axis=0)
        out_ref[...] = jax.nn.sigmoid(acc + b_ref[0, 0])

    return pl.pallas_call(
        body,
        grid=(B // blk,),
        in_specs=[
            pl.BlockSpec((blk, EMBED), lambda i: (i, 0)),
            pl.BlockSpec((3, EMBED, blk), lambda i: (0, 0, i)),
            pl.BlockSpec((1, 4 * EMBED), lambda i: (0, 0)),
            pl.BlockSpec((1, 1), lambda i: (0, 0)),
        ],
        out_specs=pl.BlockSpec((blk,), lambda i: (i,)),
        out_shape=jax.ShapeDtypeStruct((B,), jnp.float32),
        interpret=interpret,
    )(emb, sig_t, fc_w, fc_b.reshape(1, 1))


def kernel(node_idx, signal_list, node_embed, fc_w, fc_b):
    emb = _sc_gather(node_embed, node_idx.astype(jnp.int32))
    sig_t = jnp.transpose(signal_list, (0, 2, 1))
    p = _tc_dense(emb, sig_t, fc_w, fc_b)
    return p[:, None]


# speedup vs baseline: 1.7034x; 1.7034x over previous
import functools

import jax
import jax.numpy as jnp
from jax import lax
from jax.experimental import pallas as pl
from jax.experimental.pallas import tpu as pltpu
from jax.experimental.pallas import tpu_sc as plsc

N_NODES = 1000000
EMBED = 64
B = 16384


def _sc_gather(table, idx):
    info = plsc.get_sparse_core_info()
    nw = info.num_cores * info.num_subcores
    b_per_w = B // nw
    mesh = plsc.VectorSubcoreMesh(core_axis_name="c", subcore_axis_name="s")

    @functools.partial(
        pl.kernel,
        mesh=mesh,
        out_type=jax.ShapeDtypeStruct((B, EMBED), jnp.float32),
        scratch_types=[
            pltpu.VMEM((b_per_w,), jnp.int32),
            pltpu.VMEM((b_per_w, EMBED), jnp.float32),
            pltpu.SemaphoreType.DMA,
        ],
    )
    def k(table_hbm, idx_hbm, out_hbm, idx_v, rows_v, sem):
        wid = lax.axis_index("s") * info.num_cores + lax.axis_index("c")
        base = wid * b_per_w
        pltpu.sync_copy(idx_hbm.at[pl.ds(base, b_per_w)], idx_v)

        @pl.loop(0, b_per_w, step=16)
        def _issue(i0):
            vec = idx_v[pl.ds(i0, 16)]
            for j in range(16):
                pltpu.async_copy(
                    table_hbm.at[pl.ds(vec[j], 1), :],
                    rows_v.at[pl.ds(i0 + j, 1), :],
                    sem,
                )

        pltpu.make_async_copy(
            table_hbm.at[pl.ds(0, b_per_w), :], rows_v, sem
        ).wait()
        pltpu.sync_copy(rows_v, out_hbm.at[pl.ds(base, b_per_w)])

    return k(table, idx)


def _tc_dense(emb, sig_t, fc_w, fc_b, interpret=False):
    blk = 2048

    def body(emb_ref, sig_ref, w_ref, b_ref, out_ref):
        w = w_ref[...]
        acc = jnp.sum(emb_ref[...] * w[0, :EMBED][None, :], axis=1)
        st = sig_ref[...]
        for k in range(3):
            wk = w[0, (k + 1) * EMBED:(k + 2) * EMBED]
            acc = acc + jnp.sum(st[k] * wk[:, None], axis=0)
        out_ref[...] = jax.nn.sigmoid(acc + b_ref[0, 0])

    return pl.pallas_call(
        body,
        grid=(B // blk,),
        in_specs=[
            pl.BlockSpec((blk, EMBED), lambda i: (i, 0)),
            pl.BlockSpec((3, EMBED, blk), lambda i: (0, 0, i)),
            pl.BlockSpec((1, 4 * EMBED), lambda i: (0, 0)),
            pl.BlockSpec((1, 1), lambda i: (0, 0)),
        ],
        out_specs=pl.BlockSpec((blk,), lambda i: (i,)),
        out_shape=jax.ShapeDtypeStruct((B,), jnp.float32),
        interpret=interpret,
    )(emb, sig_t, fc_w, fc_b.reshape(1, 1))


def kernel(node_idx, signal_list, node_embed, fc_w, fc_b):
    emb = _sc_gather(node_embed, node_idx.astype(jnp.int32))
    sig_t = jnp.transpose(signal_list, (0, 2, 1))
    p = _tc_dense(emb, sig_t, fc_w, fc_b)
    return p[:, None]
